# Initial kernel scaffold; baseline (speedup 1.0000x reference)
import jax
import jax.numpy as jnp
from jax.experimental import pallas as pl


def kernel(x, scores, padding_mask, lens):
    raise NotImplementedError("write your pallas kernel here")


# speedup vs baseline: 19.9417x; 19.9417x over previous
import jax
import jax.numpy as jnp
from jax import lax
from jax.experimental import pallas as pl
from jax.experimental.pallas import tpu as pltpu

_MASK_RATE = 0.15
_MSK_ID = 4


def _select_body(x_ref, scores_ref, pmask_ref, lens_ref,
                 newx_ref, tmask_ref, ms_ref):
    scores = scores_ref[...]
    pmask = pmask_ref[...]
    ms = jnp.where(pmask, jnp.float32(0.0), scores)
    ms_ref[...] = ms

    u = lax.bitcast_convert_type(ms, jnp.uint32)
    neg = u >= jnp.uint32(0x80000000)
    ku = jnp.where(neg, ~u, u | jnp.uint32(0x80000000))

    lens = lens_ref[...]
    k = jnp.maximum((lens.astype(jnp.float32) * jnp.float32(_MASK_RATE))
                    .astype(jnp.int32), 1)

    def step(i, t):
        bit = jnp.uint32(31) - i.astype(jnp.uint32)
        cand = t | (jnp.uint32(1) << bit)
        cnt = jnp.sum((ku >= cand).astype(jnp.int32), axis=0, keepdims=True)
        return jnp.where(cnt >= k, cand, t)

    t0 = jnp.zeros(k.shape, jnp.uint32)
    thr = lax.fori_loop(0, 32, step, t0)

    cnt_gt = jnp.sum((ku > thr).astype(jnp.int32), axis=0, keepdims=True)
    cnt_ge = jnp.sum((ku >= thr).astype(jnp.int32), axis=0, keepdims=True)
    needed = k - cnt_gt
    eq = ku == thr

    any_dup = jnp.any(cnt_ge > k)

    def _rank_cumsum(e):
        r = e.astype(jnp.int32)
        s = r.shape[0]
        d = 1
        while d < s:
            shifted = jnp.concatenate(
                [jnp.zeros((d, r.shape[1]), jnp.int32), r[:-d, :]], axis=0)
            r = r + shifted
            d *= 2
        return r

    rank = lax.cond(any_dup, _rank_cumsum,
                    lambda e: e.astype(jnp.int32), eq)
    sel = (ku > thr) | (eq & (rank <= needed))

    newx = jnp.where(sel, jnp.int32(_MSK_ID), x_ref[...])
    newx_ref[...] = newx
    tmask_ref[...] = newx == jnp.int32(_MSK_ID)


def kernel(x, scores, padding_mask, lens):
    s, b = scores.shape
    lens2d = lens.reshape(1, b).astype(jnp.int32)
    out_shapes = (
        jax.ShapeDtypeStruct((s, b), x.dtype),
        jax.ShapeDtypeStruct((s, b), jnp.bool_),
        jax.ShapeDtypeStruct((s, b), jnp.float32),
    )
    new_x, topk_mask, masked_scores = pl.pallas_call(
        _select_body,
        out_shape=out_shapes,
    )(x, scores, padding_mask, lens2d)
    return new_x, topk_mask, masked_scores
